# baseline (speedup 1.0000x reference)
import functools

import jax
import jax.numpy as jnp
from jax import lax
from jax.experimental import pallas as pl
from jax.experimental.pallas import tpu as pltpu
from jax.experimental.pallas import tpu_sc as plsc


def _gather_kernel(B, D, b_per_w, CH, NC, interpret=False):
    n_ch = b_per_w // CH
    D2 = 2 * D

    mesh = plsc.VectorSubcoreMesh(core_axis_name="c", subcore_axis_name="s")

    @functools.partial(
        pl.kernel,
        mesh=mesh,
        out_type=jax.ShapeDtypeStruct((D, B), jnp.float32),
        scratch_types=[
            pltpu.VMEM((b_per_w,), jnp.int32),
            pltpu.VMEM((b_per_w,), jnp.int32),
            pltpu.VMEM((b_per_w, D2), jnp.float32),
            pltpu.VMEM((D, b_per_w), jnp.float32),
            pltpu.SemaphoreType.DMA,
        ],
        compiler_params=pltpu.CompilerParams(
            use_tc_tiling_on_sc=True, needs_layout_passes=False
        ),
        interpret=interpret,
    )
    def k(table2_hbm, idx_hbm, outT_hbm, idx_v, blk_v, rows_v, cols_v, sem):
        wid = lax.axis_index("s") * NC + lax.axis_index("c")
        base = wid * b_per_w
        pltpu.sync_copy(idx_hbm.at[pl.ds(base, b_per_w)], idx_v)
        for v in range(b_per_w // 16):
            iv = idx_v[pl.ds(v * 16, 16)]
            blk_v[pl.ds(v * 16, 16)] = iv >> 1
        copies = [
            pltpu.async_copy(
                table2_hbm.at[blk_v.at[pl.ds(c * CH, CH)]],
                rows_v.at[pl.ds(c * CH, CH)],
                sem,
            )
            for c in range(n_ch)
        ]
        for cp in copies:
            cp.wait()
        lanes = lax.iota(jnp.int32, 16)
        for g in range(b_per_w // 16):
            iv = idx_v[pl.ds(g * 16, 16)]
            colbase = (iv & 1) * D
            rowids = lanes + (g * 16)
            for c in range(D):
                vals = plsc.load_gather(rows_v, [rowids, colbase + c])
                cols_v[c, pl.ds(g * 16, 16)] = vals
        pltpu.sync_copy(cols_v, outT_hbm.at[:, pl.ds(base, b_per_w)])

    return k


def kernel(input, index):
    V, D = input.shape
    B = index.shape[0]
    idx32 = index.astype(jnp.int32)
    table2 = input.reshape(V // 2, 2 * D)

    info = plsc.get_sparse_core_info()
    NC, NS = info.num_cores, info.num_subcores
    NW = NC * NS
    b_per_w = B // NW

    k = _gather_kernel(B, D, b_per_w, 128, NC)
    outT = k(table2, idx32)
    return outT.T


# speedup vs baseline: 1.1070x; 1.1070x over previous
import functools

import jax
import jax.numpy as jnp
from jax import lax
from jax.experimental import pallas as pl
from jax.experimental.pallas import tpu as pltpu
from jax.experimental.pallas import tpu_sc as plsc


def _gather_kernel(B, D, b_per_w, CH, NC, interpret=False):
    n_ch = b_per_w // CH
    D2 = 2 * D

    mesh = plsc.VectorSubcoreMesh(core_axis_name="c", subcore_axis_name="s")

    @functools.partial(
        pl.kernel,
        mesh=mesh,
        out_type=jax.ShapeDtypeStruct((D, B), jnp.float32),
        scratch_types=[
            pltpu.VMEM((b_per_w,), jnp.int32),
            pltpu.VMEM((b_per_w, D2), jnp.float32),
            pltpu.VMEM((D, b_per_w), jnp.float32),
            pltpu.SemaphoreType.DMA,
        ],
        compiler_params=pltpu.CompilerParams(
            use_tc_tiling_on_sc=True, needs_layout_passes=False
        ),
        interpret=interpret,
    )
    def k(table_hbm, idx_hbm, outT_hbm, idx_v, rows_v, cols_v, sem):
        wid = lax.axis_index("s") * NC + lax.axis_index("c")
        base = wid * b_per_w
        pltpu.sync_copy(idx_hbm.at[pl.ds(base, b_per_w)], idx_v)
        copies = [
            pltpu.async_copy(
                table_hbm.at[idx_v.at[pl.ds(c * CH, CH)]],
                rows_v.at[pl.ds(c * CH, CH)],
                sem,
            )
            for c in range(n_ch)
        ]
        for cp in copies:
            cp.wait()
        lanes = lax.iota(jnp.int32, 16)
        for g in range(b_per_w // 16):
            rowids = lanes + (g * 16)
            for c in range(D):
                vals = plsc.load_gather(rows_v, [rowids, jnp.full((16,), c, jnp.int32)])
                cols_v[c, pl.ds(g * 16, 16)] = vals
        pltpu.sync_copy(cols_v, outT_hbm.at[:, pl.ds(base, b_per_w)])

    return k


def kernel(input, index):
    V, D = input.shape
    B = index.shape[0]
    idx32 = index.astype(jnp.int32)
    table_pad = jnp.pad(input, ((0, 0), (0, D)))

    info = plsc.get_sparse_core_info()
    NC, NS = info.num_cores, info.num_subcores
    NW = NC * NS
    b_per_w = B // NW

    k = _gather_kernel(B, D, b_per_w, 128, NC)
    outT = k(table_pad, idx32)
    return outT.T
